# baseline (speedup 1.0000x reference)
import functools

import jax
import jax.numpy as jnp
from jax import lax
from jax.experimental import pallas as pl
from jax.experimental.pallas import tpu as pltpu
from jax.experimental.pallas import tpu_sc as plsc

N = 10000
D = 128
E = 320000
NCLASS = 8

NC = 2
NS = 16
NW = NC * NS
K = 80
NCH = E // (NW * K)
EPW = E // NW
KS = 80
NCHS = 128
EPWP = NCHS * KS
EPAD = NW * EPWP - E

NP = 10240
ZR = 128

_mesh = plsc.VectorSubcoreMesh(core_axis_name="c", subcore_axis_name="s")


@functools.partial(
    pl.kernel,
    out_type=jax.ShapeDtypeStruct((NC * NP,), jnp.float32),
    mesh=_mesh,
    scratch_types=[
        pltpu.VMEM((NCH, K), jnp.int32),
        pltpu.VMEM((K,), jnp.float32),
        pltpu.VMEM((640,), jnp.float32),
        pltpu.VMEM_SHARED((NP,), jnp.float32),
    ],
)
def _deg_kernel(dst_hbm, out_hbm, dstv, ones_v, zv, hist_sh):
    c = lax.axis_index("c")
    s = lax.axis_index("s")
    wid = c * NS + s

    zero16 = jnp.zeros((16,), jnp.float32)
    one16 = jnp.ones((16,), jnp.float32)
    for i in range(640 // 16):
        zv[pl.ds(i * 16, 16)] = zero16
    for i in range(K // 16):
        ones_v[pl.ds(i * 16, 16)] = one16
    pltpu.sync_copy(zv, hist_sh.at[pl.ds(s * 640, 640)])
    pltpu.sync_copy(dst_hbm.at[wid], dstv)
    plsc.subcore_barrier()

    def body(j, carry):
        pltpu.sync_copy(ones_v, hist_sh.at[dstv.at[j]], add=True)
        return carry

    lax.fori_loop(0, NCH, body, 0)
    plsc.subcore_barrier()
    pltpu.sync_copy(hist_sh.at[pl.ds(s * 640, 640)],
                    out_hbm.at[pl.ds(c * NP + s * 640, 640)])


@functools.partial(
    pl.kernel,
    out_type=jax.ShapeDtypeStruct((NC * N, D), jnp.float32),
    mesh=_mesh,
    scratch_types=[
        pltpu.VMEM((NCHS, KS), jnp.int32),
        pltpu.VMEM((NCHS, KS), jnp.int32),
        pltpu.VMEM((KS, D), jnp.float32),
        pltpu.VMEM_SHARED((NP, D), jnp.float32),
        pltpu.SemaphoreType.DMA,
    ],
)
def _edge_scatter_kernel(u_hbm, src_hbm, dst_hbm, out_hbm,
                         srcv, dstv, rows, acc_sh, gsem):
    c = lax.axis_index("c")
    s = lax.axis_index("s")
    wid = c * NS + s

    zero16 = jnp.zeros((16,), jnp.float32)

    def zbody(r, carry):
        for k in range(D // 16):
            rows[r, pl.ds(k * 16, 16)] = zero16
        return carry

    lax.fori_loop(0, KS, zbody, 0)
    for t in range(NP // NS // KS):
        pltpu.sync_copy(rows, acc_sh.at[pl.ds(s * (NP // NS) + t * KS, KS)])
    pltpu.sync_copy(src_hbm.at[wid], srcv)
    pltpu.sync_copy(dst_hbm.at[wid], dstv)
    plsc.subcore_barrier()

    def body(j, carry):
        pltpu.async_copy(u_hbm.at[srcv.at[j]], rows, gsem).wait()
        pltpu.sync_copy(rows, acc_sh.at[dstv.at[j]], add=True)
        return carry

    lax.fori_loop(0, NCHS, body, 0)
    plsc.subcore_barrier()
    pltpu.sync_copy(acc_sh.at[pl.ds(s * 624, 624)],
                    out_hbm.at[pl.ds(c * N + s * 624, 624)])

    @pl.when(s == NS - 1)
    def _tail():
        pltpu.sync_copy(acc_sh.at[pl.ds(16 * 624, 16)],
                        out_hbm.at[pl.ds(c * N + 16 * 624, 16)])


R = 1000
GRID = N // R
NBLK = GRID


def _bc_body(h0_ref, h1_ref, fe_ref, w1_ref, u1_ref, dinv_ref):
    deg = h0_ref[...] + h1_ref[...] + 1.0
    dinv = lax.rsqrt(deg)
    dinv_ref[...] = dinv
    u1_ref[...] = jnp.dot(fe_ref[...], w1_ref[...],
                          preferred_element_type=jnp.float32) * dinv


def _mid_body(s1a_ref, s1b_ref, u1_ref, dinv_ref, w2_ref, b1_ref, u2_ref):
    dinv = dinv_ref[...]
    h = jax.nn.relu(dinv * (s1a_ref[...] + s1b_ref[...] + u1_ref[...])
                    + b1_ref[...])
    u2_ref[...] = jnp.dot(h, w2_ref[...],
                          preferred_element_type=jnp.float32) * dinv


def _fin_body(s2a_ref, s2b_ref, u2_ref, dinv_ref, b2_ref, wfc_ref, bfc_ref,
              h2_ref, log_ref):
    h2 = (dinv_ref[...] * (s2a_ref[...] + s2b_ref[...] + u2_ref[...])
          + b2_ref[...])
    h2_ref[...] = h2
    log_ref[...] = jnp.dot(h2, wfc_ref[...],
                           preferred_element_type=jnp.float32) + bfc_ref[...]


def _row_spec(shape):
    return pl.BlockSpec(shape, lambda i: (i, 0))


def _full_spec(shape):
    return pl.BlockSpec(shape, lambda i: (0, 0))


_bc_call = pl.pallas_call(
    _bc_body,
    grid=(GRID,),
    in_specs=[
        _row_spec((R, 1)), _row_spec((R, 1)),
        _row_spec((R, D)), _full_spec((D, D)),
    ],
    out_specs=[_row_spec((R, D)), _row_spec((R, 1))],
    out_shape=[jax.ShapeDtypeStruct((N, D), jnp.float32),
               jax.ShapeDtypeStruct((N, 1), jnp.float32)],
)

_mid_call = pl.pallas_call(
    _mid_body,
    grid=(GRID,),
    in_specs=[
        _row_spec((R, D)),
        pl.BlockSpec((R, D), lambda i: (i + NBLK, 0)),
        _row_spec((R, D)), _row_spec((R, 1)),
        _full_spec((D, D)), _full_spec((1, D)),
    ],
    out_specs=[_row_spec((R, D))],
    out_shape=[jax.ShapeDtypeStruct((N, D), jnp.float32)],
)

_fin_call = pl.pallas_call(
    _fin_body,
    grid=(GRID,),
    in_specs=[
        _row_spec((R, D)),
        pl.BlockSpec((R, D), lambda i: (i + NBLK, 0)),
        _row_spec((R, D)), _row_spec((R, 1)),
        _full_spec((1, D)), _full_spec((D, NCLASS)), _full_spec((1, NCLASS)),
    ],
    out_specs=[_row_spec((R, D)), _row_spec((R, NCLASS))],
    out_shape=[jax.ShapeDtypeStruct((N, D), jnp.float32),
               jax.ShapeDtypeStruct((N, NCLASS), jnp.float32)],
)


def kernel(x, edge_index, free_embedding, W1, b1, W2, b2, Wfc, bfc):
    pad_dst = N + (jnp.arange(EPAD, dtype=jnp.int32) % (NP - N))
    srcp = jnp.concatenate(
        [edge_index[0], jnp.zeros((EPAD,), jnp.int32)]).reshape(NW, NCHS, KS)
    dstp = jnp.concatenate(
        [edge_index[1], pad_dst]).reshape(NW, NCHS, KS)
    dst3d = edge_index[1].reshape(NW, NCH, K)

    hist = _deg_kernel(dst3d)
    h0 = hist[0:N].reshape(N, 1)
    h1 = hist[NP:NP + N].reshape(N, 1)

    u1, dinv = _bc_call(h0, h1, free_embedding, W1)
    s1 = _edge_scatter_kernel(u1, srcp, dstp)
    (u2,) = _mid_call(s1, s1, u1, dinv, W2, b1.reshape(1, D))
    s2 = _edge_scatter_kernel(u2, srcp, dstp)
    h2, logits = _fin_call(s2, s2, u2, dinv, b2.reshape(1, D),
                           Wfc, bfc.reshape(1, NCLASS))
    return (logits, h2)


# speedup vs baseline: 2.3650x; 2.3650x over previous
import functools

import jax
import jax.numpy as jnp
from jax import lax
from jax.experimental import pallas as pl
from jax.experimental.pallas import tpu as pltpu
from jax.experimental.pallas import tpu_sc as plsc

N = 10000
D = 128
E = 320000
NCLASS = 8

NC = 2
NS = 16
NW = NC * NS
K = 80
NCH = E // (NW * K)
EPW = E // NW
KS = 80
NCHS = 128
EPWP = NCHS * KS
EPAD = NW * EPWP - E

NP = 10240
ZR = 128

_mesh = plsc.VectorSubcoreMesh(core_axis_name="c", subcore_axis_name="s")


@functools.partial(
    pl.kernel,
    out_type=jax.ShapeDtypeStruct((NC * NP,), jnp.float32),
    mesh=_mesh,
    scratch_types=[
        pltpu.VMEM((NCH, K), jnp.int32),
        pltpu.VMEM((K,), jnp.float32),
        pltpu.VMEM((640,), jnp.float32),
        pltpu.VMEM_SHARED((NP,), jnp.float32),
    ],
)
def _deg_kernel(dst_hbm, out_hbm, dstv, ones_v, zv, hist_sh):
    c = lax.axis_index("c")
    s = lax.axis_index("s")
    wid = c * NS + s

    zero16 = jnp.zeros((16,), jnp.float32)
    one16 = jnp.ones((16,), jnp.float32)
    for i in range(640 // 16):
        zv[pl.ds(i * 16, 16)] = zero16
    for i in range(K // 16):
        ones_v[pl.ds(i * 16, 16)] = one16
    pltpu.sync_copy(zv, hist_sh.at[pl.ds(s * 640, 640)])
    pltpu.sync_copy(dst_hbm.at[wid], dstv)
    plsc.subcore_barrier()

    def body(j, carry):
        pltpu.sync_copy(ones_v, hist_sh.at[dstv.at[j]], add=True)
        return carry

    lax.fori_loop(0, NCH, body, 0)
    plsc.subcore_barrier()
    pltpu.sync_copy(hist_sh.at[pl.ds(s * 640, 640)],
                    out_hbm.at[pl.ds(c * NP + s * 640, 640)])


@functools.partial(
    pl.kernel,
    out_type=jax.ShapeDtypeStruct((NC * N, D), jnp.float32),
    mesh=_mesh,
    scratch_types=[
        pltpu.VMEM((NCHS, KS), jnp.int32),
        pltpu.VMEM((NCHS, KS), jnp.int32),
        pltpu.VMEM((KS, D), jnp.float32),
        pltpu.VMEM_SHARED((NP, D), jnp.float32),
        pltpu.SemaphoreType.DMA,
    ],
)
def _edge_scatter_kernel(u_hbm, src_hbm, dst_hbm, out_hbm,
                         srcv, dstv, rows, acc_sh, gsem):
    c = lax.axis_index("c")
    s = lax.axis_index("s")
    wid = c * NS + s

    zero16 = jnp.zeros((16,), jnp.float32)

    def zbody(r, carry):
        for k in range(D // 16):
            rows[r, pl.ds(k * 16, 16)] = zero16
        return carry

    lax.fori_loop(0, KS, zbody, 0)
    for t in range(NP // NS // KS):
        pltpu.sync_copy(rows, acc_sh.at[pl.ds(s * (NP // NS) + t * KS, KS)])
    pltpu.sync_copy(src_hbm.at[wid], srcv)
    pltpu.sync_copy(dst_hbm.at[wid], dstv)
    plsc.subcore_barrier()

    def body(j, carry):
        pltpu.async_copy(u_hbm.at[srcv.at[j]], rows, gsem).wait()
        pltpu.sync_copy(rows, acc_sh.at[dstv.at[j]], add=True)
        return carry

    lax.fori_loop(0, NCHS, body, 0)
    plsc.subcore_barrier()
    pltpu.sync_copy(acc_sh.at[pl.ds(s * 624, 624)],
                    out_hbm.at[pl.ds(c * N + s * 624, 624)])

    @pl.when(s == NS - 1)
    def _tail():
        pltpu.sync_copy(acc_sh.at[pl.ds(16 * 624, 16)],
                        out_hbm.at[pl.ds(c * N + 16 * 624, 16)])


R = 1000
GRID = N // R
NBLK = GRID


def _bc_body(h0_ref, h1_ref, fe_ref, w1_ref, u1_ref, dinv_ref):
    deg = h0_ref[...] + h1_ref[...] + 1.0
    dinv = lax.rsqrt(deg)
    dinv_ref[...] = dinv
    u1_ref[...] = jnp.dot(fe_ref[...], w1_ref[...],
                          preferred_element_type=jnp.float32) * dinv


def _mid_body(s1a_ref, s1b_ref, u1_ref, dinv_ref, w2_ref, b1_ref, u2_ref):
    dinv = dinv_ref[...]
    h = jax.nn.relu(dinv * (s1a_ref[...] + s1b_ref[...] + u1_ref[...])
                    + b1_ref[...])
    u2_ref[...] = jnp.dot(h, w2_ref[...],
                          preferred_element_type=jnp.float32) * dinv


def _fin_body(s2a_ref, s2b_ref, u2_ref, dinv_ref, b2_ref, wfc_ref, bfc_ref,
              h2_ref, log_ref):
    h2 = (dinv_ref[...] * (s2a_ref[...] + s2b_ref[...] + u2_ref[...])
          + b2_ref[...])
    h2_ref[...] = h2
    log_ref[...] = jnp.dot(h2, wfc_ref[...],
                           preferred_element_type=jnp.float32) + bfc_ref[...]


def _row_spec(shape):
    return pl.BlockSpec(shape, lambda i: (i, 0))


def _full_spec(shape):
    return pl.BlockSpec(shape, lambda i: (0, 0))


_bc_call = pl.pallas_call(
    _bc_body,
    grid=(GRID,),
    in_specs=[
        _row_spec((R, 1)), _row_spec((R, 1)),
        _row_spec((R, D)), _full_spec((D, D)),
    ],
    out_specs=[_row_spec((R, D)), _row_spec((R, 1))],
    out_shape=[jax.ShapeDtypeStruct((N, D), jnp.float32),
               jax.ShapeDtypeStruct((N, 1), jnp.float32)],
)

_mid_call = pl.pallas_call(
    _mid_body,
    grid=(GRID,),
    in_specs=[
        _row_spec((R, D)),
        pl.BlockSpec((R, D), lambda i: (i + NBLK, 0)),
        _row_spec((R, D)), _row_spec((R, 1)),
        _full_spec((D, D)), _full_spec((1, D)),
    ],
    out_specs=[_row_spec((R, D))],
    out_shape=[jax.ShapeDtypeStruct((N, D), jnp.float32)],
)

_fin_call = pl.pallas_call(
    _fin_body,
    grid=(GRID,),
    in_specs=[
        _row_spec((R, D)),
        pl.BlockSpec((R, D), lambda i: (i + NBLK, 0)),
        _row_spec((R, D)), _row_spec((R, 1)),
        _full_spec((1, D)), _full_spec((D, NCLASS)), _full_spec((1, NCLASS)),
    ],
    out_specs=[_row_spec((R, D)), _row_spec((R, NCLASS))],
    out_shape=[jax.ShapeDtypeStruct((N, D), jnp.float32),
               jax.ShapeDtypeStruct((N, NCLASS), jnp.float32)],
)


def kernel(x, edge_index, free_embedding, W1, b1, W2, b2, Wfc, bfc):
    pad_iota = jnp.arange(EPAD, dtype=jnp.int32)
    pad_src = (pad_iota * 79) % N
    pad_dst = N + (pad_iota % (NP - N))
    srcp = jnp.concatenate(
        [edge_index[0], pad_src]).reshape(NW, NCHS, KS)
    dstp = jnp.concatenate(
        [edge_index[1], pad_dst]).reshape(NW, NCHS, KS)
    dst3d = edge_index[1].reshape(NW, NCH, K)

    hist = _deg_kernel(dst3d)
    h0 = hist[0:N].reshape(N, 1)
    h1 = hist[NP:NP + N].reshape(N, 1)

    u1, dinv = _bc_call(h0, h1, free_embedding, W1)
    s1 = _edge_scatter_kernel(u1, srcp, dstp)
    (u2,) = _mid_call(s1, s1, u1, dinv, W2, b1.reshape(1, D))
    s2 = _edge_scatter_kernel(u2, srcp, dstp)
    h2, logits = _fin_call(s2, s2, u2, dinv, b2.reshape(1, D),
                           Wfc, bfc.reshape(1, NCLASS))
    return (logits, h2)


# speedup vs baseline: 3.8975x; 1.6480x over previous
import functools

import jax
import jax.numpy as jnp
from jax import lax
from jax.experimental import pallas as pl
from jax.experimental.pallas import tpu as pltpu
from jax.experimental.pallas import tpu_sc as plsc

N = 10000
D = 128
E = 320000
NCLASS = 8

NC = 2
NS = 16
NW = NC * NS
K = 80
NCH = E // (NW * K)
EPW = E // NW
KS = 128
NCHS = 80
IB = 8
NB = NCHS // IB
EPWP = NCHS * KS
EPAD = NW * EPWP - E

NP = 10240
ZR = 128

_mesh = plsc.VectorSubcoreMesh(core_axis_name="c", subcore_axis_name="s")


@functools.partial(
    pl.kernel,
    out_type=jax.ShapeDtypeStruct((NC * NP,), jnp.float32),
    mesh=_mesh,
    scratch_types=[
        pltpu.VMEM((NCH, K), jnp.int32),
        pltpu.VMEM((K,), jnp.float32),
        pltpu.VMEM((640,), jnp.float32),
        pltpu.VMEM_SHARED((NP,), jnp.float32),
    ],
)
def _deg_kernel(dst_hbm, out_hbm, dstv, ones_v, zv, hist_sh):
    c = lax.axis_index("c")
    s = lax.axis_index("s")
    wid = c * NS + s

    zero16 = jnp.zeros((16,), jnp.float32)
    one16 = jnp.ones((16,), jnp.float32)
    for i in range(640 // 16):
        zv[pl.ds(i * 16, 16)] = zero16
    for i in range(K // 16):
        ones_v[pl.ds(i * 16, 16)] = one16
    pltpu.sync_copy(zv, hist_sh.at[pl.ds(s * 640, 640)])
    pltpu.sync_copy(dst_hbm.at[wid], dstv)
    plsc.subcore_barrier()

    def body(j, carry):
        pltpu.sync_copy(ones_v, hist_sh.at[dstv.at[j]], add=True)
        return carry

    lax.fori_loop(0, NCH, body, 0)
    plsc.subcore_barrier()
    pltpu.sync_copy(hist_sh.at[pl.ds(s * 640, 640)],
                    out_hbm.at[pl.ds(c * NP + s * 640, 640)])


@functools.partial(
    pl.kernel,
    out_type=jax.ShapeDtypeStruct((NC * N, D), jnp.float32),
    mesh=_mesh,
    scratch_types=[
        pltpu.VMEM((IB, KS), jnp.int32),
        pltpu.VMEM((IB, KS), jnp.int32),
        pltpu.VMEM((IB, KS), jnp.int32),
        pltpu.VMEM((IB, KS), jnp.int32),
        pltpu.VMEM((KS, D), jnp.float32),
        pltpu.VMEM((KS, D), jnp.float32),
        pltpu.VMEM_SHARED((NP, D), jnp.float32),
        pltpu.SemaphoreType.DMA,
        pltpu.SemaphoreType.DMA,
        pltpu.SemaphoreType.DMA,
        pltpu.SemaphoreType.DMA,
    ],
)
def _edge_scatter_kernel(u_hbm, src_hbm, dst_hbm, out_hbm,
                         sb0, sb1, db0, db1, rows0, rows1, acc_sh,
                         gsem0, gsem1, isem0, isem1):
    c = lax.axis_index("c")
    s = lax.axis_index("s")
    wid = c * NS + s
    base = wid * NCHS

    zero16 = jnp.zeros((16,), jnp.float32)

    def zbody(r, carry):
        for k in range(D // 16):
            rows0[r, pl.ds(k * 16, 16)] = zero16
        return carry

    lax.fori_loop(0, KS, zbody, 0)
    for t in range(NP // NS // KS):
        pltpu.sync_copy(rows0, acc_sh.at[pl.ds(s * (NP // NS) + t * KS, KS)])
    pltpu.sync_copy(src_hbm.at[pl.ds(pl.multiple_of(base, IB), IB)], sb0)
    pltpu.sync_copy(dst_hbm.at[pl.ds(pl.multiple_of(base, IB), IB)], db0)
    plsc.subcore_barrier()

    pltpu.async_copy(u_hbm.at[sb0.at[0]], rows0, gsem0)
    pltpu.async_copy(u_hbm.at[sb0.at[1]], rows1, gsem1)

    def _block(b, sbX, dbX, sbY, dbY):
        for i in range(IB):
            rp, gp = (rows0, gsem0) if i % 2 == 0 else (rows1, gsem1)
            pltpu.make_async_copy(u_hbm.at[sbX.at[i]], rp, gp).wait()
            pltpu.sync_copy(rp, acc_sh.at[dbX.at[i]], add=True)
            if i == 0:
                @pl.when(b + 1 < NB)
                def _ld():
                    off = pl.multiple_of(base + (b + 1) * IB, IB)
                    pltpu.async_copy(src_hbm.at[pl.ds(off, IB)], sbY, isem0)
                    pltpu.async_copy(dst_hbm.at[pl.ds(off, IB)], dbY, isem1)
            if i == IB - 3:
                @pl.when(b + 1 < NB)
                def _wld():
                    pltpu.make_async_copy(
                        src_hbm.at[pl.ds(0, IB)], sbY, isem0).wait()
                    pltpu.make_async_copy(
                        dst_hbm.at[pl.ds(0, IB)], dbY, isem1).wait()
            if i < IB - 2:
                pltpu.async_copy(u_hbm.at[sbX.at[i + 2]], rp, gp)
            else:
                @pl.when(b + 1 < NB)
                def _fn():
                    pltpu.async_copy(u_hbm.at[sbY.at[i - (IB - 2)]], rp, gp)

    def outer(bb, carry):
        _block(2 * bb, sb0, db0, sb1, db1)
        _block(2 * bb + 1, sb1, db1, sb0, db0)
        return carry

    lax.fori_loop(0, NB // 2, outer, 0)
    plsc.subcore_barrier()
    pltpu.sync_copy(acc_sh.at[pl.ds(s * 624, 624)],
                    out_hbm.at[pl.ds(c * N + s * 624, 624)])

    @pl.when(s == NS - 1)
    def _tail():
        pltpu.sync_copy(acc_sh.at[pl.ds(16 * 624, 16)],
                        out_hbm.at[pl.ds(c * N + 16 * 624, 16)])


R = 1000
GRID = N // R
NBLK = GRID


def _bc_body(h0_ref, h1_ref, fe_ref, w1_ref, u1_ref, dinv_ref):
    deg = h0_ref[...] + h1_ref[...] + 1.0
    dinv = lax.rsqrt(deg)
    dinv_ref[...] = dinv
    u1_ref[...] = jnp.dot(fe_ref[...], w1_ref[...],
                          preferred_element_type=jnp.float32) * dinv


def _mid_body(s1a_ref, s1b_ref, u1_ref, dinv_ref, w2_ref, b1_ref, u2_ref):
    dinv = dinv_ref[...]
    h = jax.nn.relu(dinv * (s1a_ref[...] + s1b_ref[...] + u1_ref[...])
                    + b1_ref[...])
    u2_ref[...] = jnp.dot(h, w2_ref[...],
                          preferred_element_type=jnp.float32) * dinv


def _fin_body(s2a_ref, s2b_ref, u2_ref, dinv_ref, b2_ref, wfc_ref, bfc_ref,
              h2_ref, log_ref):
    h2 = (dinv_ref[...] * (s2a_ref[...] + s2b_ref[...] + u2_ref[...])
          + b2_ref[...])
    h2_ref[...] = h2
    log_ref[...] = jnp.dot(h2, wfc_ref[...],
                           preferred_element_type=jnp.float32) + bfc_ref[...]


def _row_spec(shape):
    return pl.BlockSpec(shape, lambda i: (i, 0))


def _full_spec(shape):
    return pl.BlockSpec(shape, lambda i: (0, 0))


_bc_call = pl.pallas_call(
    _bc_body,
    grid=(GRID,),
    in_specs=[
        _row_spec((R, 1)), _row_spec((R, 1)),
        _row_spec((R, D)), _full_spec((D, D)),
    ],
    out_specs=[_row_spec((R, D)), _row_spec((R, 1))],
    out_shape=[jax.ShapeDtypeStruct((N, D), jnp.float32),
               jax.ShapeDtypeStruct((N, 1), jnp.float32)],
)

_mid_call = pl.pallas_call(
    _mid_body,
    grid=(GRID,),
    in_specs=[
        _row_spec((R, D)),
        pl.BlockSpec((R, D), lambda i: (i + NBLK, 0)),
        _row_spec((R, D)), _row_spec((R, 1)),
        _full_spec((D, D)), _full_spec((1, D)),
    ],
    out_specs=[_row_spec((R, D))],
    out_shape=[jax.ShapeDtypeStruct((N, D), jnp.float32)],
)

_fin_call = pl.pallas_call(
    _fin_body,
    grid=(GRID,),
    in_specs=[
        _row_spec((R, D)),
        pl.BlockSpec((R, D), lambda i: (i + NBLK, 0)),
        _row_spec((R, D)), _row_spec((R, 1)),
        _full_spec((1, D)), _full_spec((D, NCLASS)), _full_spec((1, NCLASS)),
    ],
    out_specs=[_row_spec((R, D)), _row_spec((R, NCLASS))],
    out_shape=[jax.ShapeDtypeStruct((N, D), jnp.float32),
               jax.ShapeDtypeStruct((N, NCLASS), jnp.float32)],
)


def kernel(x, edge_index, free_embedding, W1, b1, W2, b2, Wfc, bfc):
    pad_iota = jnp.arange(EPAD, dtype=jnp.int32)
    pad_src = (pad_iota * 79) % N
    pad_dst = N + (pad_iota % (NP - N))
    srcp = jnp.concatenate(
        [edge_index[0], pad_src]).reshape(NW * NCHS, KS)
    dstp = jnp.concatenate(
        [edge_index[1], pad_dst]).reshape(NW * NCHS, KS)
    dst3d = edge_index[1].reshape(NW, NCH, K)

    hist = _deg_kernel(dst3d)
    h0 = hist[0:N].reshape(N, 1)
    h1 = hist[NP:NP + N].reshape(N, 1)

    u1, dinv = _bc_call(h0, h1, free_embedding, W1)
    s1 = _edge_scatter_kernel(u1, srcp, dstp)
    (u2,) = _mid_call(s1, s1, u1, dinv, W2, b1.reshape(1, D))
    s2 = _edge_scatter_kernel(u2, srcp, dstp)
    h2, logits = _fin_call(s2, s2, u2, dinv, b2.reshape(1, D),
                           Wfc, bfc.reshape(1, NCLASS))
    return (logits, h2)
